# baseline (speedup 1.0000x reference)
import functools
import jax
import jax.numpy as jnp
from jax import lax
from jax.experimental import pallas as pl
from jax.experimental.pallas import tpu as pltpu
from jax.experimental.pallas import tpu_sc as plsc

_VOCAB = 50265
_H = 768
_PAD = 1
_MAX_POS = 514
_EPS = 1e-12
_B, _S = 1024, 200

_L = 16
_NC, _NS = 2, 16
_NW = _NC * _NS
_ROWS_W = _B // _NW
_SPAD = 208
_NCH16 = _SPAD // _L
_CH = 64
_NFULL = _S // _CH
_TAIL = _S - _NFULL * _CH
_NJ = _H // _L


def _rsqrt16(v):
    i = plsc.bitcast(v, jnp.int32)
    i = jnp.int32(0x5F3759DF) - lax.shift_right_arithmetic(i, 1)
    y = plsc.bitcast(i, jnp.float32)
    for _ in range(3):
        y = y * (1.5 - 0.5 * v * y * y)
    return y


def _splat(s, dtype):
    return lax.broadcast_in_dim(s, (_L,), ()).astype(dtype)


def _emb_body(ids_hbm, word_hbm, pos_hbm, tt_hbm, g_hbm, beta_hbm, out_hbm,
              ids_v, pos_v, wrows, prows, tt_v, g_v, b_v, sem_w, sem_p):
    wid = lax.axis_index("s") * _NC + lax.axis_index("c")
    base_row = wid * _ROWS_W

    pltpu.sync_copy(tt_hbm.at[0], tt_v)
    pltpu.sync_copy(g_hbm, g_v)
    pltpu.sync_copy(beta_hbm, b_v)

    pad_vec = jnp.full((_L,), _PAD, dtype=jnp.int32)

    def row_pos(r, _):
        ids_v[r, pl.ds(_SPAD - _L, _L)] = pad_vec
        pltpu.sync_copy(ids_hbm.at[base_row + r], ids_v.at[r, pl.ds(0, _S)])

        def chunk(j, carry):
            v = ids_v[r, pl.ds(j * _L, _L)]
            m = (v != _PAD).astype(jnp.int32)
            cs = lax.cumsum(m, axis=0)
            pos = (cs + carry) * m + _PAD
            pos_v[r, pl.ds(j * _L, _L)] = pos
            return carry + _splat(jnp.sum(m), jnp.int32)

        lax.fori_loop(0, _NCH16, chunk, jnp.zeros((_L,), jnp.int32))
        return 0

    lax.fori_loop(0, _ROWS_W, row_pos, 0)

    inv_h = 1.0 / _H

    def do_chunk(r, c0, n_tok):
        b = base_row + r
        cw = pltpu.async_copy(word_hbm.at[ids_v.at[r, pl.ds(c0, n_tok)]],
                              wrows.at[pl.ds(0, n_tok)], sem_w)
        cp = pltpu.async_copy(pos_hbm.at[pos_v.at[r, pl.ds(c0, n_tok)]],
                              prows.at[pl.ds(0, n_tok)], sem_p)
        cw.wait()
        cp.wait()

        def tok(t, _):
            s = jnp.zeros((_L,), jnp.float32)
            q = jnp.zeros((_L,), jnp.float32)
            for j in range(_NJ):
                x = (wrows[t, pl.ds(j * _L, _L)]
                     + prows[t, pl.ds(j * _L, _L)]
                     + tt_v[pl.ds(j * _L, _L)])
                wrows[t, pl.ds(j * _L, _L)] = x
                s = s + x
                q = q + x * x
            mean = _splat(jnp.sum(s), jnp.float32) * inv_h
            var = _splat(jnp.sum(q), jnp.float32) * inv_h - mean * mean
            inv = _rsqrt16(var + _EPS)
            for j in range(_NJ):
                x = wrows[t, pl.ds(j * _L, _L)]
                wrows[t, pl.ds(j * _L, _L)] = (
                    (x - mean) * inv * g_v[pl.ds(j * _L, _L)]
                    + b_v[pl.ds(j * _L, _L)])
            return 0

        lax.fori_loop(0, n_tok, tok, 0)
        pltpu.sync_copy(wrows.at[pl.ds(0, n_tok)],
                        out_hbm.at[b, pl.ds(c0, n_tok)])

    def row_work(r, _):
        def full(c, __):
            do_chunk(r, c * _CH, _CH)
            return 0
        lax.fori_loop(0, _NFULL, full, 0)
        do_chunk(r, _NFULL * _CH, _TAIL)
        return 0

    lax.fori_loop(0, _ROWS_W, row_work, 0)


@functools.partial(jax.jit, donate_argnums=())
def kernel(input_ids, word_table, pos_table, tt_table, gamma, beta):
    mesh = plsc.VectorSubcoreMesh(core_axis_name="c", subcore_axis_name="s",
                                  num_cores=_NC, num_subcores=_NS)
    run = pl.kernel(
        _emb_body,
        out_type=jax.ShapeDtypeStruct((_B, _S, _H), jnp.float32),
        mesh=mesh,
        scratch_types=[
            pltpu.VMEM((_ROWS_W, _SPAD), jnp.int32),
            pltpu.VMEM((_ROWS_W, _SPAD), jnp.int32),
            pltpu.VMEM((_CH, _H), jnp.float32),
            pltpu.VMEM((_CH, _H), jnp.float32),
            pltpu.VMEM((_H,), jnp.float32),
            pltpu.VMEM((_H,), jnp.float32),
            pltpu.VMEM((_H,), jnp.float32),
            pltpu.SemaphoreType.DMA,
            pltpu.SemaphoreType.DMA,
        ],
        compiler_params=pltpu.CompilerParams(use_tc_tiling_on_sc=False,
                                             needs_layout_passes=False),
        name="clap_text_embeddings_sc",
    )
    return run(input_ids.astype(jnp.int32), word_table, pos_table,
               tt_table, gamma, beta)
